# baseline (speedup 1.0000x reference)
import functools

import jax
import jax.numpy as jnp
from jax import lax
from jax.experimental import pallas as pl
from jax.experimental.pallas import tpu as pltpu
from jax.experimental.pallas import tpu_sc as plsc

D = 64
VOCAB = 1000000
B = 16384
P = 8
NC = 2
NS = 16
LANES = 16
NW = NC * NS
BPW = B // NW
GROUPS = BPW // LANES

TC_BLK = 4096
SPLIT = 94 * TC_BLK

TLIM = (VOCAB // 128) * 128
NTAIL = VOCAB - TLIM
CHW = 512
SC_RANGE = TLIM - SPLIT
NCHUNK_SC = SC_RANGE // CHW
CPW = -(-NCHUNK_SC // NW)
BLKS = CHW // (16 * LANES)

_mesh = plsc.VectorSubcoreMesh(
    core_axis_name="c", subcore_axis_name="s", num_cores=NC, num_subcores=NS
)


def _project_tc_body(t_ref, w_ref, b_ref, o_ref):
    o_ref[...] = (
        lax.dot_general(
            t_ref[...], w_ref[...],
            dimension_numbers=(((0,), (0,)), ((), ())),
            preferred_element_type=jnp.float32,
        )
        + b_ref[...]
    )


_project_tc = pl.pallas_call(
    _project_tc_body,
    grid=(SPLIT // TC_BLK,),
    in_specs=[
        pl.BlockSpec((D, TC_BLK), lambda i: (0, i)),
        pl.BlockSpec((D, P), lambda i: (0, 0)),
        pl.BlockSpec((1, P), lambda i: (0, 0)),
    ],
    out_specs=pl.BlockSpec((TC_BLK, P), lambda i: (i, 0)),
    out_shape=jax.ShapeDtypeStruct((SPLIT, P), jnp.float32),
)


@functools.partial(
    pl.kernel,
    out_type=jax.ShapeDtypeStruct(((SC_RANGE + NTAIL) * P,), jnp.float32),
    mesh=_mesh,
    scratch_types=[
        pltpu.VMEM((D, CHW), jnp.float32),
        pltpu.VMEM((D, CHW), jnp.float32),
        pltpu.VMEM((CHW * P,), jnp.float32),
        pltpu.VMEM((D, NTAIL), jnp.float32),
        pltpu.VMEM((2, D * LANES), jnp.float32),
        pltpu.VMEM((2, LANES), jnp.float32),
        pltpu.SemaphoreType.DMA,
        pltpu.SemaphoreType.DMA,
    ],
    compiler_params=pltpu.CompilerParams(needs_layout_passes=False),
)
def _project_sc(table_hbm, tail_hbm, w_hbm, b_hbm, p8_hbm,
                buf_a, buf_b, stage_v, tail_v, w_v, b_v, sem_a, sem_b):
    wid = lax.axis_index("s") * NC + lax.axis_index("c")

    pltpu.sync_copy(w_hbm, w_v)
    pltpu.sync_copy(b_hbm, b_v)

    iota = lax.iota(jnp.int32, LANES)
    zeros16 = jnp.zeros((LANES,), jnp.float32)
    zeros_i = jnp.zeros((LANES,), jnp.int32)
    ones_i = jnp.full((LANES,), 1, jnp.int32)
    b0 = b_v[0]
    b1 = b_v[1]

    def chunk_g(i):
        return jnp.minimum(wid * CPW + i, NCHUNK_SC - 1)

    def fire(i, buf, sem):
        off = pl.multiple_of(SPLIT + chunk_g(i) * CHW, 128)
        pltpu.async_copy(table_hbm.at[:, pl.ds(off, CHW)], buf, sem)

    def wait_chunk(buf, sem):
        pltpu.make_async_copy(table_hbm.at[:, pl.ds(0, CHW)], buf, sem).wait()

    def project_chunk(i, buf):
        out_base = chunk_g(i) * CHW

        def block(blk, carry):
            base = blk * 16 * LANES

            def dstep(d, accs):
                w0 = w_v[0, pl.ds(d * LANES, LANES)]
                w1 = w_v[1, pl.ds(d * LANES, LANES)]
                new = []
                for g8 in range(16):
                    col = buf[d, pl.ds(base + g8 * LANES, LANES)]
                    new.append(accs[2 * g8] + col * w0)
                    new.append(accs[2 * g8 + 1] + col * w1)
                return tuple(new)

            accs = lax.fori_loop(0, D, dstep, (zeros16,) * 32, unroll=4)
            for g8 in range(16):
                tok = base + g8 * LANES + iota
                plsc.store_scatter(stage_v, [tok * P], accs[2 * g8] + b0)
                plsc.store_scatter(stage_v, [tok * P + 1], accs[2 * g8 + 1] + b1)
            return carry

        lax.fori_loop(0, BLKS, block, 0)
        pltpu.sync_copy(stage_v, p8_hbm.at[pl.ds(out_base * P, CHW * P)])

    fire(0, buf_a, sem_a)
    fire(1, buf_b, sem_b)

    def pairbody(k, carry):
        i = 2 * k
        wait_chunk(buf_a, sem_a)
        project_chunk(i, buf_a)
        fire(i + 2, buf_a, sem_a)
        wait_chunk(buf_b, sem_b)
        project_chunk(i + 1, buf_b)
        fire(i + 3, buf_b, sem_b)
        return carry

    lax.fori_loop(0, CPW // 2, pairbody, 0)
    wait_chunk(buf_a, sem_a)
    wait_chunk(buf_b, sem_b)

    @pl.when(wid == 0)
    def _():
        pltpu.sync_copy(tail_hbm, tail_v)

        def dstep_t(d, accs):
            w0 = w_v[0, pl.ds(d * LANES, LANES)]
            w1 = w_v[1, pl.ds(d * LANES, LANES)]
            new = []
            for g8 in range(4):
                col = tail_v[d, pl.ds(g8 * LANES, LANES)]
                new.append(accs[2 * g8] + col * w0)
                new.append(accs[2 * g8 + 1] + col * w1)
            return tuple(new)

        accs = lax.fori_loop(0, D, dstep_t, (zeros16,) * 8, unroll=8)
        for g8 in range(4):
            tok = g8 * LANES + iota
            plsc.store_scatter(stage_v, [tok * P], accs[2 * g8] + b0)
            plsc.store_scatter(stage_v, [tok * P + 1], accs[2 * g8 + 1] + b1)
        pltpu.sync_copy(stage_v.at[pl.ds(0, NTAIL * P)],
                        p8_hbm.at[pl.ds(SC_RANGE * P, NTAIL * P)])


@functools.partial(
    pl.kernel,
    out_type=jax.ShapeDtypeStruct((2, B), jnp.float32),
    mesh=_mesh,
    scratch_types=[
        pltpu.VMEM((BPW,), jnp.int32),
        pltpu.VMEM((BPW, P), jnp.float32),
        pltpu.VMEM((BPW * P,), jnp.float32),
        pltpu.VMEM((BPW,), jnp.float32),
        pltpu.VMEM((BPW,), jnp.float32),
        pltpu.SemaphoreType.DMA,
        pltpu.SemaphoreType.DMA,
    ],
    compiler_params=pltpu.CompilerParams(needs_layout_passes=False),
)
def _gather_sc(tok_hbm, p8tc_hbm, p8sc_hbm, out_hbm,
               raw_v, rows_tc, rows_sc, out0_v, out1_v, sem_tc, sem_sc):
    wid = lax.axis_index("s") * NC + lax.axis_index("c")

    pltpu.sync_copy(tok_hbm.at[wid], raw_v)

    iota = lax.iota(jnp.int32, LANES)
    zeros_i = jnp.zeros((LANES,), jnp.int32)
    ones_i = jnp.full((LANES,), 1, jnp.int32)

    def fire(g, cnt_tc):
        toks = raw_v[pl.ds(g * LANES, LANES)]
        n_tc = plsc.all_reduce_population_count(toks < SPLIT)[0]
        for l in range(LANES):
            t = toks[l]
            slot = g * LANES + l

            @pl.when(t < SPLIT)
            def _():
                pltpu.async_copy(
                    p8tc_hbm.at[pl.ds(t, 1)],
                    rows_tc.at[pl.ds(slot, 1)], sem_tc)

            @pl.when(t >= SPLIT)
            def _():
                pltpu.async_copy(
                    p8sc_hbm.at[pl.ds((t - SPLIT) * P, P)],
                    rows_sc.at[pl.ds(slot * P, P)], sem_sc)

        return cnt_tc + n_tc

    total_tc = lax.fori_loop(0, GROUPS, fire, jnp.int32(0))

    def drain_tc(i, carry):
        pltpu.make_async_copy(
            p8tc_hbm.at[pl.ds(0, 1)], rows_tc.at[pl.ds(0, 1)], sem_tc
        ).wait()
        return carry

    def drain_sc(i, carry):
        pltpu.make_async_copy(
            p8sc_hbm.at[pl.ds(0, P)], rows_sc.at[pl.ds(0, P)], sem_sc
        ).wait()
        return carry

    lax.fori_loop(0, total_tc, drain_tc, 0)
    lax.fori_loop(0, BPW - total_tc, drain_sc, 0)

    def group(g, carry):
        row_idx = g * LANES + iota
        t_vec = raw_v[pl.ds(g * LANES, LANES)]
        from_tc = t_vec < SPLIT
        a0 = jnp.where(from_tc,
                       plsc.load_gather(rows_tc, [row_idx, zeros_i]),
                       plsc.load_gather(rows_sc, [row_idx * P]))
        a1 = jnp.where(from_tc,
                       plsc.load_gather(rows_tc, [row_idx, ones_i]),
                       plsc.load_gather(rows_sc, [row_idx * P + 1]))
        out0_v[pl.ds(g * LANES, LANES)] = a0
        out1_v[pl.ds(g * LANES, LANES)] = a1
        return carry

    lax.fori_loop(0, GROUPS, group, 0)

    base = pl.multiple_of(wid * BPW, 128)
    pltpu.sync_copy(out0_v, out_hbm.at[0, pl.ds(base, BPW)])
    pltpu.sync_copy(out1_v, out_hbm.at[1, pl.ds(base, BPW)])


def kernel(ids, token_emb, fc_w, fc_b):
    tok = ids[:, 0].astype(jnp.int32).reshape(NW, BPW)
    table_t = token_emb.T
    tail_t = table_t[:, TLIM:]
    w8 = jnp.zeros((D, P), jnp.float32).at[:, :2].set(fc_w.T)
    b8 = jnp.zeros((1, P), jnp.float32).at[0, :2].set(fc_b)
    w_bcast = jnp.broadcast_to(fc_w[:, :, None], (2, D, LANES)).reshape(2, D * LANES)
    b_bcast = jnp.broadcast_to(fc_b[:, None], (2, LANES))
    p8_tc = _project_tc(table_t, w8, b8)
    p8_sc = _project_sc(table_t, tail_t, w_bcast, b_bcast)
    return _gather_sc(tok, p8_tc, p8_sc).T


# speedup vs baseline: 1.0046x; 1.0046x over previous
import functools

import jax
import jax.numpy as jnp
from jax import lax
from jax.experimental import pallas as pl
from jax.experimental.pallas import tpu as pltpu
from jax.experimental.pallas import tpu_sc as plsc

D = 64
VOCAB = 1000000
B = 16384
P = 8
NC = 2
NS = 16
LANES = 16
NW = NC * NS
BPW = B // NW
GROUPS = BPW // LANES

TC_BLK = 4096
SPLIT = 94 * TC_BLK

TLIM = (VOCAB // 128) * 128
NTAIL = VOCAB - TLIM
CHW = 512
SC_RANGE = TLIM - SPLIT
NCHUNK_SC = SC_RANGE // CHW
CPW = -(-NCHUNK_SC // NW)
BLKS = CHW // (8 * LANES)

_mesh = plsc.VectorSubcoreMesh(
    core_axis_name="c", subcore_axis_name="s", num_cores=NC, num_subcores=NS
)


def _project_tc_body(t_ref, w_ref, b_ref, o_ref):
    o_ref[...] = (
        lax.dot_general(
            t_ref[...], w_ref[...],
            dimension_numbers=(((0,), (0,)), ((), ())),
            preferred_element_type=jnp.float32,
        )
        + b_ref[...]
    )


_project_tc = pl.pallas_call(
    _project_tc_body,
    grid=(SPLIT // TC_BLK,),
    in_specs=[
        pl.BlockSpec((D, TC_BLK), lambda i: (0, i)),
        pl.BlockSpec((D, P), lambda i: (0, 0)),
        pl.BlockSpec((1, P), lambda i: (0, 0)),
    ],
    out_specs=pl.BlockSpec((TC_BLK, P), lambda i: (i, 0)),
    out_shape=jax.ShapeDtypeStruct((SPLIT, P), jnp.float32),
)


@functools.partial(
    pl.kernel,
    out_type=jax.ShapeDtypeStruct(((SC_RANGE + NTAIL) * P,), jnp.float32),
    mesh=_mesh,
    scratch_types=[
        pltpu.VMEM((D, CHW), jnp.float32),
        pltpu.VMEM((D, CHW), jnp.float32),
        pltpu.VMEM((CHW * P,), jnp.float32),
        pltpu.VMEM((D, NTAIL), jnp.float32),
        pltpu.VMEM((2, D * LANES), jnp.float32),
        pltpu.VMEM((2, LANES), jnp.float32),
        pltpu.SemaphoreType.DMA,
        pltpu.SemaphoreType.DMA,
    ],
    compiler_params=pltpu.CompilerParams(needs_layout_passes=False),
)
def _project_sc(table_hbm, tail_hbm, w_hbm, b_hbm, p8_hbm,
                buf_a, buf_b, stage_v, tail_v, w_v, b_v, sem_a, sem_b):
    wid = lax.axis_index("s") * NC + lax.axis_index("c")

    pltpu.sync_copy(w_hbm, w_v)
    pltpu.sync_copy(b_hbm, b_v)

    iota = lax.iota(jnp.int32, LANES)
    zeros16 = jnp.zeros((LANES,), jnp.float32)
    zeros_i = jnp.zeros((LANES,), jnp.int32)
    ones_i = jnp.full((LANES,), 1, jnp.int32)
    b0 = b_v[0]
    b1 = b_v[1]

    def chunk_g(i):
        return jnp.minimum(wid * CPW + i, NCHUNK_SC - 1)

    def fire(i, buf, sem):
        off = pl.multiple_of(SPLIT + chunk_g(i) * CHW, 128)
        pltpu.async_copy(table_hbm.at[:, pl.ds(off, CHW)], buf, sem)

    def wait_chunk(buf, sem):
        pltpu.make_async_copy(table_hbm.at[:, pl.ds(0, CHW)], buf, sem).wait()

    def project_chunk(i, buf):
        out_base = chunk_g(i) * CHW

        def block(blk, carry):
            base = blk * 8 * LANES

            def dstep(d, accs):
                w0 = w_v[0, pl.ds(d * LANES, LANES)]
                w1 = w_v[1, pl.ds(d * LANES, LANES)]
                new = []
                for g8 in range(8):
                    col = buf[d, pl.ds(base + g8 * LANES, LANES)]
                    new.append(accs[2 * g8] + col * w0)
                    new.append(accs[2 * g8 + 1] + col * w1)
                return tuple(new)

            accs = lax.fori_loop(0, D, dstep, (zeros16,) * 16, unroll=8)
            for g8 in range(8):
                tok = base + g8 * LANES + iota
                plsc.store_scatter(stage_v, [tok * P], accs[2 * g8] + b0)
                plsc.store_scatter(stage_v, [tok * P + 1], accs[2 * g8 + 1] + b1)
            return carry

        lax.fori_loop(0, BLKS, block, 0)
        pltpu.sync_copy(stage_v, p8_hbm.at[pl.ds(out_base * P, CHW * P)])

    fire(0, buf_a, sem_a)
    fire(1, buf_b, sem_b)

    def pairbody(k, carry):
        i = 2 * k
        wait_chunk(buf_a, sem_a)
        project_chunk(i, buf_a)
        fire(i + 2, buf_a, sem_a)
        wait_chunk(buf_b, sem_b)
        project_chunk(i + 1, buf_b)
        fire(i + 3, buf_b, sem_b)
        return carry

    lax.fori_loop(0, CPW // 2, pairbody, 0)
    wait_chunk(buf_a, sem_a)
    wait_chunk(buf_b, sem_b)

    @pl.when(wid == 0)
    def _():
        pltpu.sync_copy(tail_hbm, tail_v)

        def dstep_t(d, accs):
            w0 = w_v[0, pl.ds(d * LANES, LANES)]
            w1 = w_v[1, pl.ds(d * LANES, LANES)]
            new = []
            for g8 in range(4):
                col = tail_v[d, pl.ds(g8 * LANES, LANES)]
                new.append(accs[2 * g8] + col * w0)
                new.append(accs[2 * g8 + 1] + col * w1)
            return tuple(new)

        accs = lax.fori_loop(0, D, dstep_t, (zeros16,) * 8, unroll=8)
        for g8 in range(4):
            tok = g8 * LANES + iota
            plsc.store_scatter(stage_v, [tok * P], accs[2 * g8] + b0)
            plsc.store_scatter(stage_v, [tok * P + 1], accs[2 * g8 + 1] + b1)
        pltpu.sync_copy(stage_v.at[pl.ds(0, NTAIL * P)],
                        p8_hbm.at[pl.ds(SC_RANGE * P, NTAIL * P)])


@functools.partial(
    pl.kernel,
    out_type=jax.ShapeDtypeStruct((2, B), jnp.float32),
    mesh=_mesh,
    scratch_types=[
        pltpu.VMEM((BPW,), jnp.int32),
        pltpu.VMEM((BPW, P), jnp.float32),
        pltpu.VMEM((BPW * P,), jnp.float32),
        pltpu.VMEM((BPW,), jnp.float32),
        pltpu.VMEM((BPW,), jnp.float32),
        pltpu.SemaphoreType.DMA,
        pltpu.SemaphoreType.DMA,
    ],
    compiler_params=pltpu.CompilerParams(needs_layout_passes=False),
)
def _gather_sc(tok_hbm, p8tc_hbm, p8sc_hbm, out_hbm,
               raw_v, rows_tc, rows_sc, out0_v, out1_v, sem_tc, sem_sc):
    wid = lax.axis_index("s") * NC + lax.axis_index("c")

    pltpu.sync_copy(tok_hbm.at[wid], raw_v)

    iota = lax.iota(jnp.int32, LANES)
    zeros_i = jnp.zeros((LANES,), jnp.int32)
    ones_i = jnp.full((LANES,), 1, jnp.int32)

    def fire(g, cnt_tc):
        toks = raw_v[pl.ds(g * LANES, LANES)]
        n_tc = plsc.all_reduce_population_count(toks < SPLIT)[0]
        for l in range(LANES):
            t = toks[l]
            slot = g * LANES + l

            @pl.when(t < SPLIT)
            def _():
                pltpu.async_copy(
                    p8tc_hbm.at[pl.ds(t, 1)],
                    rows_tc.at[pl.ds(slot, 1)], sem_tc)

            @pl.when(t >= SPLIT)
            def _():
                pltpu.async_copy(
                    p8sc_hbm.at[pl.ds((t - SPLIT) * P, P)],
                    rows_sc.at[pl.ds(slot * P, P)], sem_sc)

        return cnt_tc + n_tc

    total_tc = lax.fori_loop(0, GROUPS, fire, jnp.int32(0))

    def drain_tc(i, carry):
        pltpu.make_async_copy(
            p8tc_hbm.at[pl.ds(0, 1)], rows_tc.at[pl.ds(0, 1)], sem_tc
        ).wait()
        return carry

    def drain_sc(i, carry):
        pltpu.make_async_copy(
            p8sc_hbm.at[pl.ds(0, P)], rows_sc.at[pl.ds(0, P)], sem_sc
        ).wait()
        return carry

    lax.fori_loop(0, total_tc, drain_tc, 0)
    lax.fori_loop(0, BPW - total_tc, drain_sc, 0)

    def group(g, carry):
        row_idx = g * LANES + iota
        t_vec = raw_v[pl.ds(g * LANES, LANES)]
        from_tc = t_vec < SPLIT
        a0 = jnp.where(from_tc,
                       plsc.load_gather(rows_tc, [row_idx, zeros_i]),
                       plsc.load_gather(rows_sc, [row_idx * P]))
        a1 = jnp.where(from_tc,
                       plsc.load_gather(rows_tc, [row_idx, ones_i]),
                       plsc.load_gather(rows_sc, [row_idx * P + 1]))
        out0_v[pl.ds(g * LANES, LANES)] = a0
        out1_v[pl.ds(g * LANES, LANES)] = a1
        return carry

    lax.fori_loop(0, GROUPS, group, 0)

    base = pl.multiple_of(wid * BPW, 128)
    pltpu.sync_copy(out0_v, out_hbm.at[0, pl.ds(base, BPW)])
    pltpu.sync_copy(out1_v, out_hbm.at[1, pl.ds(base, BPW)])


def kernel(ids, token_emb, fc_w, fc_b):
    tok = ids[:, 0].astype(jnp.int32).reshape(NW, BPW)
    table_t = token_emb.T
    tail_t = table_t[:, TLIM:]
    w8 = jnp.zeros((D, P), jnp.float32).at[:, :2].set(fc_w.T)
    b8 = jnp.zeros((1, P), jnp.float32).at[0, :2].set(fc_b)
    w_bcast = jnp.broadcast_to(fc_w[:, :, None], (2, D, LANES)).reshape(2, D * LANES)
    b_bcast = jnp.broadcast_to(fc_b[:, None], (2, LANES))
    p8_tc = _project_tc(table_t, w8, b8)
    p8_sc = _project_sc(table_t, tail_t, w_bcast, b_bcast)
    return _gather_sc(tok, p8_tc, p8_sc).T
